# baseline (speedup 1.0000x reference)
import functools

import jax
import jax.numpy as jnp
from jax import lax
from jax.experimental import pallas as pl
from jax.experimental.pallas import tpu as pltpu
from jax.experimental.pallas import tpu_sc as plsc

DIM = 64
ROW = 128
LANES = 16
NUM_CORES = 2
NUM_SUBCORES = 16
NUM_WORKERS = NUM_CORES * NUM_SUBCORES


def _repack_kernel(vocab: int):
    n_full = vocab // ROW
    rem = vocab - n_full * ROW
    n_blocks = n_full + (1 if rem else 0)
    iters = (n_blocks + NUM_WORKERS - 1) // NUM_WORKERS
    mesh = plsc.VectorSubcoreMesh(core_axis_name="c", subcore_axis_name="s")

    @functools.partial(
        pl.kernel,
        out_type=(
            jax.ShapeDtypeStruct((vocab, ROW), jnp.float32),
            jax.ShapeDtypeStruct((vocab, ROW), jnp.float32),
        ),
        mesh=mesh,
        scratch_types=[
            pltpu.VMEM((DIM, ROW), jnp.float32),
            pltpu.VMEM((ROW, ROW), jnp.float32),
            pltpu.VMEM((DIM, DIM), jnp.float32),
            pltpu.VMEM((DIM, ROW), jnp.float32),
        ],
        compiler_params=pltpu.CompilerParams(
            use_tc_tiling_on_sc=True, needs_layout_passes=False
        ),
    )
    def kern(ttT_hbm, ctT_hbm, t_out, c_out, vin, vout, vin_r, vout_r):
        wid = lax.axis_index("s") * NUM_CORES + lax.axis_index("c")
        dvecs = [16 * k + lax.iota(jnp.int32, LANES) for k in range(DIM // LANES)]

        def transpose_block(src_hbm, dst_hbm, v0):
            pltpu.sync_copy(src_hbm.at[:, pl.ds(v0, ROW)], vin)

            def quad(q, _):
                for sub in range(4):
                    vv = q * 4 + sub
                    splat = jnp.full((LANES,), vv, jnp.int32)
                    for k in range(DIM // LANES):
                        g = plsc.load_gather(vin, [dvecs[k], splat])
                        vout[vv, pl.ds(k * LANES, LANES)] = g
                return 0

            lax.fori_loop(0, ROW // 4, quad, 0)
            pltpu.sync_copy(vout, dst_hbm.at[pl.ds(v0, ROW), :])

        def transpose_rem(src_hbm, dst_hbm):
            v0 = n_full * ROW
            pltpu.sync_copy(src_hbm.at[:, pl.ds(v0, rem)], vin_r)

            def quad(q, _):
                for sub in range(4):
                    vv = q * 4 + sub
                    splat = jnp.full((LANES,), vv, jnp.int32)
                    for k in range(DIM // LANES):
                        g = plsc.load_gather(vin_r, [dvecs[k], splat])
                        vout_r[vv, pl.ds(k * LANES, LANES)] = g
                return 0

            lax.fori_loop(0, rem // 4, quad, 0)
            pltpu.sync_copy(vout_r, dst_hbm.at[pl.ds(v0, rem), :])

        def body(i, _):
            b = wid + NUM_WORKERS * i

            @pl.when(b < n_full)
            def _():
                transpose_block(ttT_hbm, t_out, b * ROW)
                transpose_block(ctT_hbm, c_out, b * ROW)

            if rem:
                @pl.when(b == n_full)
                def _():
                    transpose_rem(ttT_hbm, t_out)
                    transpose_rem(ctT_hbm, c_out)
            return 0

        lax.fori_loop(0, iters, body, 0)

    return kern


def _sc_dot_kernel(n_total: int, chunk: int):
    per_w = n_total // NUM_WORKERS
    n_chunks = per_w // chunk
    assert n_chunks % 2 == 0
    mesh = plsc.VectorSubcoreMesh(core_axis_name="c", subcore_axis_name="s")

    @functools.partial(
        pl.kernel,
        out_type=jax.ShapeDtypeStruct((n_total,), jnp.float32),
        mesh=mesh,
        scratch_types=[
            pltpu.VMEM((chunk,), jnp.int32),
            pltpu.VMEM((chunk,), jnp.int32),
            pltpu.VMEM((chunk,), jnp.int32),
            pltpu.VMEM((chunk,), jnp.int32),
            pltpu.VMEM((chunk, ROW), jnp.float32),
            pltpu.VMEM((chunk, ROW), jnp.float32),
            pltpu.VMEM((chunk, ROW), jnp.float32),
            pltpu.VMEM((chunk, ROW), jnp.float32),
            pltpu.VMEM((chunk,), jnp.float32),
            pltpu.VMEM((chunk,), jnp.float32),
            pltpu.SemaphoreType.DMA,
            pltpu.SemaphoreType.DMA,
        ],
        compiler_params=pltpu.CompilerParams(
            use_tc_tiling_on_sc=True, needs_layout_passes=False
        ),
    )
    def kern(tgt_hbm, ctx_hbm, tt_hbm, ct_hbm, out_hbm,
             idx_t0, idx_t1, idx_c0, idx_c1,
             rows_t0, rows_t1, rows_c0, rows_c1,
             out_v0, out_v1, sem0, sem1):
        idx_t = (idx_t0, idx_t1)
        idx_c = (idx_c0, idx_c1)
        rows_t = (rows_t0, rows_t1)
        rows_c = (rows_c0, rows_c1)
        out_v = (out_v0, out_v1)
        sem = (sem0, sem1)
        wid = lax.axis_index("s") * NUM_CORES + lax.axis_index("c")
        wbase = wid * per_w

        def fire(slot, base):
            pltpu.sync_copy(tgt_hbm.at[pl.ds(base, chunk)], idx_t[slot])
            pltpu.sync_copy(ctx_hbm.at[pl.ds(base, chunk)], idx_c[slot])
            pltpu.async_copy(tt_hbm.at[idx_t[slot]], rows_t[slot], sem[slot])
            pltpu.async_copy(ct_hbm.at[idx_c[slot]], rows_c[slot], sem[slot])

        def drain(slot):
            pltpu.make_async_copy(
                tt_hbm.at[idx_t[slot]], rows_t[slot], sem[slot]).wait()
            pltpu.make_async_copy(
                ct_hbm.at[idx_c[slot]], rows_c[slot], sem[slot]).wait()

        def compute(slot, base):
            rt, rc, ov = rows_t[slot], rows_c[slot], out_v[slot]
            lanes = lax.iota(jnp.int32, LANES)

            def group_body(i, _):
                r0 = i * LANES
                out_acc = jnp.zeros((LANES,), jnp.float32)
                for rr in range(LANES):
                    r = r0 + rr
                    prods = [
                        rt[r, pl.ds(k * LANES, LANES)]
                        * rc[r, pl.ds(k * LANES, LANES)]
                        for k in range(DIM // LANES)
                    ]
                    s = (prods[0] + prods[1]) + (prods[2] + prods[3])
                    tot = jnp.sum(s)
                    out_acc = jnp.where(lanes == rr,
                                        jnp.full((LANES,), tot), out_acc)
                ov[pl.ds(r0, LANES)] = out_acc
                return 0

            lax.fori_loop(0, chunk // LANES, group_body, 0)
            pltpu.sync_copy(ov, out_hbm.at[pl.ds(base, chunk)])

        fire(0, wbase)

        def body(kk, _):
            c0 = wbase + (2 * kk) * chunk
            c1 = c0 + chunk
            fire(1, c1)
            drain(0)
            compute(0, c0)

            @pl.when(2 * kk + 2 < n_chunks)
            def _():
                fire(0, c1 + chunk)

            drain(1)
            compute(1, c1)
            return 0

        lax.fori_loop(0, n_chunks // 2, body, 0)

    return kern


def kernel(target, context, target_table, context_table):
    b, l = target.shape
    n_total = b * l
    vocab = target_table.shape[0]
    tgt = target.reshape(n_total).astype(jnp.int32)
    ctx = context.reshape(n_total).astype(jnp.int32)
    t128, c128 = _repack_kernel(vocab)(target_table.T, context_table.T)
    sim = _sc_dot_kernel(n_total, chunk=128)(tgt, ctx, t128, c128)
    return sim.reshape(b, l)


# speedup vs baseline: 3.3707x; 3.3707x over previous
import functools

import jax
import jax.numpy as jnp
from jax import lax
from jax.experimental import pallas as pl
from jax.experimental.pallas import tpu as pltpu
from jax.experimental.pallas import tpu_sc as plsc

DIM = 64
ROW = 128
LANES = 16
NUM_CORES = 2
NUM_SUBCORES = 16
NUM_WORKERS = NUM_CORES * NUM_SUBCORES


def _sc_dot_kernel(n_total: int, chunk: int):
    per_w = n_total // NUM_WORKERS
    n_chunks = per_w // chunk
    assert n_chunks % 2 == 0
    mesh = plsc.VectorSubcoreMesh(core_axis_name="c", subcore_axis_name="s")

    @functools.partial(
        pl.kernel,
        out_type=jax.ShapeDtypeStruct((n_total,), jnp.float32),
        mesh=mesh,
        scratch_types=[
            pltpu.VMEM((chunk,), jnp.int32),
            pltpu.VMEM((chunk,), jnp.int32),
            pltpu.VMEM((chunk,), jnp.int32),
            pltpu.VMEM((chunk,), jnp.int32),
            pltpu.VMEM((chunk, ROW), jnp.float32),
            pltpu.VMEM((chunk, ROW), jnp.float32),
            pltpu.VMEM((chunk, ROW), jnp.float32),
            pltpu.VMEM((chunk, ROW), jnp.float32),
            pltpu.VMEM((chunk,), jnp.float32),
            pltpu.VMEM((chunk,), jnp.float32),
            pltpu.SemaphoreType.DMA,
            pltpu.SemaphoreType.DMA,
        ],
        compiler_params=pltpu.CompilerParams(
            use_tc_tiling_on_sc=True, needs_layout_passes=False
        ),
    )
    def kern(tgt_hbm, ctx_hbm, tt_hbm, ct_hbm, out_hbm,
             idx_t0, idx_t1, idx_c0, idx_c1,
             rows_t0, rows_t1, rows_c0, rows_c1,
             out_v0, out_v1, sem0, sem1):
        idx_t = (idx_t0, idx_t1)
        idx_c = (idx_c0, idx_c1)
        rows_t = (rows_t0, rows_t1)
        rows_c = (rows_c0, rows_c1)
        out_v = (out_v0, out_v1)
        sem = (sem0, sem1)
        wid = lax.axis_index("s") * NUM_CORES + lax.axis_index("c")
        wbase = wid * per_w

        def fire(slot, base):
            pltpu.sync_copy(tgt_hbm.at[pl.ds(base, chunk)], idx_t[slot])
            pltpu.sync_copy(ctx_hbm.at[pl.ds(base, chunk)], idx_c[slot])
            pltpu.async_copy(tt_hbm.at[idx_t[slot]], rows_t[slot], sem[slot])
            pltpu.async_copy(ct_hbm.at[idx_c[slot]], rows_c[slot], sem[slot])

        def drain(slot):
            pltpu.make_async_copy(
                tt_hbm.at[idx_t[slot]], rows_t[slot], sem[slot]).wait()
            pltpu.make_async_copy(
                ct_hbm.at[idx_c[slot]], rows_c[slot], sem[slot]).wait()

        def compute(slot, base):
            rt, rc, ov = rows_t[slot], rows_c[slot], out_v[slot]
            lanes = lax.iota(jnp.int32, LANES)

            def group_body(i, _):
                r0 = i * LANES
                out_acc = jnp.zeros((LANES,), jnp.float32)
                for rr in range(LANES):
                    r = r0 + rr
                    prods = [
                        rt[r, pl.ds(k * LANES, LANES)]
                        * rc[r, pl.ds(k * LANES, LANES)]
                        for k in range(DIM // LANES)
                    ]
                    s = (prods[0] + prods[1]) + (prods[2] + prods[3])
                    tot = jnp.sum(s)
                    out_acc = jnp.where(lanes == rr,
                                        jnp.full((LANES,), tot), out_acc)
                ov[pl.ds(r0, LANES)] = out_acc
                return 0

            lax.fori_loop(0, chunk // LANES, group_body, 0)
            pltpu.sync_copy(ov, out_hbm.at[pl.ds(base, chunk)])

        fire(0, wbase)

        def body(kk, _):
            c0 = wbase + (2 * kk) * chunk
            c1 = c0 + chunk
            fire(1, c1)
            drain(0)
            compute(0, c0)

            @pl.when(2 * kk + 2 < n_chunks)
            def _():
                fire(0, c1 + chunk)

            drain(1)
            compute(1, c1)
            return 0

        lax.fori_loop(0, n_chunks // 2, body, 0)

    return kern


def kernel(target, context, target_table, context_table):
    b, l = target.shape
    n_total = b * l
    tgt = target.reshape(n_total).astype(jnp.int32)
    ctx = context.reshape(n_total).astype(jnp.int32)
    tt2 = jnp.pad(target_table, ((0, 0), (0, ROW - DIM)))
    ct2 = jnp.pad(context_table, ((0, 0), (0, ROW - DIM)))
    sim = _sc_dot_kernel(n_total, chunk=160)(tgt, ctx, tt2, ct2)
    return sim.reshape(b, l)


# speedup vs baseline: 3.7529x; 1.1134x over previous
import functools

import jax
import jax.numpy as jnp
from jax import lax
from jax.experimental import pallas as pl
from jax.experimental.pallas import tpu as pltpu
from jax.experimental.pallas import tpu_sc as plsc

DIM = 64
ROW = 128
LANES = 16
NUM_CORES = 2
NUM_SUBCORES = 16
NUM_WORKERS = NUM_CORES * NUM_SUBCORES


def _sc_dot_kernel(n_total: int, chunk: int):
    per_w = n_total // NUM_WORKERS
    n_chunks = per_w // chunk
    assert n_chunks % 2 == 0
    mesh = plsc.VectorSubcoreMesh(core_axis_name="c", subcore_axis_name="s")

    @functools.partial(
        pl.kernel,
        out_type=jax.ShapeDtypeStruct((n_total,), jnp.float32),
        mesh=mesh,
        scratch_types=[
            pltpu.VMEM((chunk,), jnp.int32),
            pltpu.VMEM((chunk,), jnp.int32),
            pltpu.VMEM((chunk,), jnp.int32),
            pltpu.VMEM((chunk,), jnp.int32),
            pltpu.VMEM((chunk, ROW), jnp.float32),
            pltpu.VMEM((chunk, ROW), jnp.float32),
            pltpu.VMEM((chunk, ROW), jnp.float32),
            pltpu.VMEM((chunk, ROW), jnp.float32),
            pltpu.VMEM((chunk,), jnp.float32),
            pltpu.VMEM((chunk,), jnp.float32),
            pltpu.SemaphoreType.DMA,
            pltpu.SemaphoreType.DMA,
        ],
        compiler_params=pltpu.CompilerParams(
            use_tc_tiling_on_sc=True, needs_layout_passes=False
        ),
    )
    def kern(tgt_hbm, ctx_hbm, tt_hbm, out_hbm,
             idx_t0, idx_t1, idx_c0, idx_c1,
             rows_t0, rows_t1, rows_c0, rows_c1,
             out_v0, out_v1, sem0, sem1):
        idx_t = (idx_t0, idx_t1)
        idx_c = (idx_c0, idx_c1)
        rows_t = (rows_t0, rows_t1)
        rows_c = (rows_c0, rows_c1)
        out_v = (out_v0, out_v1)
        sem = (sem0, sem1)
        wid = lax.axis_index("s") * NUM_CORES + lax.axis_index("c")
        wbase = wid * per_w

        def fire(slot, base):
            pltpu.sync_copy(tgt_hbm.at[pl.ds(base, chunk)], idx_t[slot])
            pltpu.sync_copy(ctx_hbm.at[pl.ds(base, chunk)], idx_c[slot])
            pltpu.async_copy(tt_hbm.at[idx_t[slot]], rows_t[slot], sem[slot])
            pltpu.async_copy(tt_hbm.at[idx_c[slot]], rows_c[slot], sem[slot])

        def drain(slot):
            pltpu.make_async_copy(
                tt_hbm.at[idx_t[slot]], rows_t[slot], sem[slot]).wait()
            pltpu.make_async_copy(
                tt_hbm.at[idx_c[slot]], rows_c[slot], sem[slot]).wait()

        def compute(slot, base):
            rt, rc, ov = rows_t[slot], rows_c[slot], out_v[slot]
            lanes = lax.iota(jnp.int32, LANES)

            def group_body(i, _):
                r0 = i * LANES
                out_acc = jnp.zeros((LANES,), jnp.float32)
                for rr in range(LANES):
                    r = r0 + rr
                    prods = [
                        rt[r, pl.ds(k * LANES, LANES)]
                        * rc[r, pl.ds(DIM + k * LANES, LANES)]
                        for k in range(DIM // LANES)
                    ]
                    s = (prods[0] + prods[1]) + (prods[2] + prods[3])
                    tot = jnp.sum(s)
                    out_acc = jnp.where(lanes == rr,
                                        jnp.full((LANES,), tot), out_acc)
                ov[pl.ds(r0, LANES)] = out_acc
                return 0

            lax.fori_loop(0, chunk // LANES, group_body, 0)
            pltpu.sync_copy(ov, out_hbm.at[pl.ds(base, chunk)])

        fire(0, wbase)

        def body(kk, _):
            c0 = wbase + (2 * kk) * chunk
            c1 = c0 + chunk
            fire(1, c1)
            drain(0)
            compute(0, c0)

            @pl.when(2 * kk + 2 < n_chunks)
            def _():
                fire(0, c1 + chunk)

            drain(1)
            compute(1, c1)
            return 0

        lax.fori_loop(0, n_chunks // 2, body, 0)

    return kern


def kernel(target, context, target_table, context_table):
    b, l = target.shape
    n_total = b * l
    tgt = target.reshape(n_total).astype(jnp.int32)
    ctx = context.reshape(n_total).astype(jnp.int32)
    t2 = jnp.concatenate([target_table, context_table], axis=1)
    sim = _sc_dot_kernel(n_total, chunk=160)(tgt, ctx, t2)
    return sim.reshape(b, l)
